# baseline (speedup 1.0000x reference)
import functools
import jax
import jax.numpy as jnp
from jax import lax
from jax.experimental import pallas as pl
from jax.experimental.pallas import tpu as pltpu
from jax.experimental.pallas import tpu_sc as plsc

_B = 4
_S = 8192
_H = 1024
_NW = 32
_CH = 32
_RPW = _S // _NW
_NCH = _RPW // _CH


def _sc_body(table_hbm, out_hbm, buf, in_sem, out_sem):
    wid = lax.axis_index("s") * 2 + lax.axis_index("c")
    base = wid * _RPW

    def in_copy(c):
        slot = c % 2
        return pltpu.make_async_copy(
            table_hbm.at[pl.ds(base + c * _CH, _CH), :], buf.at[slot], in_sem
        )

    def out_copy(c, b):
        slot = c % 2
        return pltpu.make_async_copy(
            buf.at[slot], out_hbm.at[b, pl.ds(base + c * _CH, _CH), :], out_sem
        )

    in_copy(0).start()
    for c in range(_NCH):
        if c + 1 < _NCH:
            if c >= 1:
                for b in range(_B):
                    out_copy(c - 1, b).wait()
            in_copy(c + 1).start()
        in_copy(c).wait()
        for b in range(_B):
            out_copy(c, b).start()
    for c in (_NCH - 2, _NCH - 1):
        for b in range(_B):
            out_copy(c, b).wait()


def kernel(x, pos_embedding):
    mesh = plsc.VectorSubcoreMesh(core_axis_name="c", subcore_axis_name="s")
    k = functools.partial(
        pl.kernel,
        mesh=mesh,
        out_type=jax.ShapeDtypeStruct((_B, _S, _H), jnp.float32),
        scratch_types=[
            pltpu.VMEM((2, _CH, _H), jnp.float32),
            pltpu.SemaphoreType.DMA,
            pltpu.SemaphoreType.DMA,
        ],
    )(_sc_body)
    return k(pos_embedding)
